# baseline (speedup 1.0000x reference)
import jax
import jax.numpy as jnp
from jax import lax
from jax.experimental import pallas as pl
from jax.experimental.pallas import tpu as pltpu
from jax.experimental.pallas import tpu_sc as plsc

_BN = 512
_BN2 = 2048
_NC = 2
_NS = 16


def _topk_kernel(src_ref, cent_ref, idx_ref, sqr_ref):
    i = pl.program_id(0)
    cent = cent_ref[...]

    @pl.when(i == 0)
    def _():
        sqr_ref[...] = jnp.sum(cent * cent, axis=0, keepdims=True)

    s = src_ref[...]
    inner = jax.lax.dot_general(
        s, cent, (((0,), (0,)), ((), ())), preferred_element_type=jnp.float32
    )
    sel = inner * 2.0 - sqr_ref[...]

    bn, kc = sel.shape
    iota = jax.lax.broadcasted_iota(jnp.int32, (bn, kc), 1)
    idxs = []
    for j in range(4):
        idx = jnp.argmax(sel, axis=1)
        idxs.append(idx)
        if j < 3:
            sel = jnp.where(iota == idx[:, None], -jnp.inf, sel)
    idx_ref[...] = jnp.stack(idxs, axis=0)


def _sc_gather_kernel(cent_hbm, idx_hbm, out_hbm, idx_v, rows_v, sem):
    wid = lax.axis_index("s") * _NC + lax.axis_index("c")
    per_w = idx_hbm.shape[0] // (_NC * _NS)
    base = wid * per_w
    pltpu.sync_copy(idx_hbm.at[pl.ds(base, per_w)], idx_v)
    pltpu.async_copy(cent_hbm.at[idx_v], rows_v, sem).wait()
    pltpu.sync_copy(rows_v, out_hbm.at[pl.ds(base, per_w)])


def _combine_kernel(g_ref, out_ref):
    g = g_ref[...]
    s = (g[0] + g[1] + g[2] + g[3]) * 0.25
    out_ref[...] = s.T


@jax.jit
def _run(source, centroids):
    B, C, N = source.shape
    Kc = centroids.shape[2]
    NB = N // _BN
    cent = centroids[0]
    cent_rows = jnp.transpose(cent)

    mesh = plsc.VectorSubcoreMesh(
        core_axis_name="c", subcore_axis_name="s",
        num_cores=_NC, num_subcores=_NS,
    )
    sc_gather = pl.kernel(
        _sc_gather_kernel,
        out_type=jax.ShapeDtypeStruct((4 * N, C), jnp.float32),
        mesh=mesh,
        scratch_types=[
            pltpu.VMEM((4 * N // (_NC * _NS),), jnp.int32),
            pltpu.VMEM((4 * N // (_NC * _NS), C), jnp.float32),
            pltpu.SemaphoreType.DMA,
        ],
    )

    topk = pl.pallas_call(
        _topk_kernel,
        grid=(NB,),
        in_specs=[
            pl.BlockSpec((C, _BN), lambda i: (0, i)),
            pl.BlockSpec((C, Kc), lambda i: (0, 0)),
        ],
        out_specs=pl.BlockSpec((4, _BN), lambda i: (0, i)),
        out_shape=jax.ShapeDtypeStruct((4, N), jnp.int32),
        scratch_shapes=[pltpu.VMEM((1, Kc), jnp.float32)],
    )

    combine = pl.pallas_call(
        _combine_kernel,
        grid=(N // _BN2,),
        in_specs=[
            pl.BlockSpec((4, _BN2, C), lambda i: (0, i, 0)),
        ],
        out_specs=pl.BlockSpec((C, _BN2), lambda i: (0, i)),
        out_shape=jax.ShapeDtypeStruct((C, N), jnp.float32),
    )

    idxs = [topk(source[b], cent) for b in range(B)]
    gs = [sc_gather(cent_rows, ib.reshape(-1)) for ib in idxs]
    outs = [combine(g.reshape(4, N, C)) for g in gs]
    return jnp.stack(outs, axis=0)


def kernel(source, centroids, k):
    return _run(source, centroids)


# speedup vs baseline: 1.0237x; 1.0237x over previous
import jax
import jax.numpy as jnp
from jax import lax
from jax.experimental import pallas as pl
from jax.experimental.pallas import tpu as pltpu
from jax.experimental.pallas import tpu_sc as plsc

_BN = 512
_BN2 = 1024
_NC = 2
_NS = 16


def _topk_kernel(src_ref, cent_ref, idx_ref, sqr_ref):
    i = pl.program_id(0)
    cent = cent_ref[...]

    @pl.when(i == 0)
    def _():
        sqr_ref[...] = jnp.sum(cent * cent, axis=0, keepdims=True)

    s = src_ref[...]
    inner = jax.lax.dot_general(
        s, cent, (((0,), (0,)), ((), ())), preferred_element_type=jnp.float32
    )
    sel = inner * 2.0 - sqr_ref[...]

    bn, kc = sel.shape
    iota = jax.lax.broadcasted_iota(jnp.int32, (bn, kc), 1)
    idxs = []
    for j in range(4):
        idx = jnp.argmax(sel, axis=1)
        idxs.append(idx)
        if j < 3:
            sel = jnp.where(iota == idx[:, None], -jnp.inf, sel)
    idx_ref[...] = jnp.stack(idxs, axis=0)


def _sc_gather_kernel(cent_hbm, idx_hbm, out_hbm, idx_v, rows_v, sem):
    wid = lax.axis_index("s") * _NC + lax.axis_index("c")
    per_w = idx_hbm.shape[0] // (_NC * _NS)
    base = wid * per_w
    pltpu.sync_copy(idx_hbm.at[pl.ds(base, per_w)], idx_v)
    pltpu.async_copy(cent_hbm.at[idx_v], rows_v, sem).wait()
    pltpu.sync_copy(rows_v, out_hbm.at[pl.ds(base, per_w)])


def _combine_kernel(g_ref, out_ref):
    g = g_ref[...]
    s = (g[0] + g[1] + g[2] + g[3]) * 0.25
    out_ref[...] = s.T


@jax.jit
def _run(source, centroids):
    B, C, N = source.shape
    Kc = centroids.shape[2]
    NB = N // _BN
    cent = centroids[0]
    cent_rows = jnp.transpose(cent)

    mesh = plsc.VectorSubcoreMesh(
        core_axis_name="c", subcore_axis_name="s",
        num_cores=_NC, num_subcores=_NS,
    )
    sc_gather = pl.kernel(
        _sc_gather_kernel,
        out_type=jax.ShapeDtypeStruct((4 * N, C), jnp.float32),
        mesh=mesh,
        scratch_types=[
            pltpu.VMEM((4 * N // (_NC * _NS),), jnp.int32),
            pltpu.VMEM((4 * N // (_NC * _NS), C), jnp.float32),
            pltpu.SemaphoreType.DMA,
        ],
    )

    topk = pl.pallas_call(
        _topk_kernel,
        grid=(NB,),
        in_specs=[
            pl.BlockSpec((C, _BN), lambda i: (0, i)),
            pl.BlockSpec((C, Kc), lambda i: (0, 0)),
        ],
        out_specs=pl.BlockSpec((4, _BN), lambda i: (0, i)),
        out_shape=jax.ShapeDtypeStruct((4, N), jnp.int32),
        scratch_shapes=[pltpu.VMEM((1, Kc), jnp.float32)],
    )

    combine = pl.pallas_call(
        _combine_kernel,
        grid=(N // _BN2,),
        in_specs=[
            pl.BlockSpec((4, _BN2, C), lambda i: (0, i, 0)),
        ],
        out_specs=pl.BlockSpec((C, _BN2), lambda i: (0, i)),
        out_shape=jax.ShapeDtypeStruct((C, N), jnp.float32),
    )

    idxs = [topk(source[b], cent) for b in range(B)]
    gs = [sc_gather(cent_rows, ib.reshape(-1)) for ib in idxs]
    outs = [combine(g.reshape(4, N, C)) for g in gs]
    return jnp.stack(outs, axis=0)


def kernel(source, centroids, k):
    return _run(source, centroids)
